# baseline (speedup 1.0000x reference)
import functools

import jax
import jax.numpy as jnp
from jax import lax
from jax.experimental import pallas as pl
from jax.experimental.pallas import tpu as pltpu
from jax.experimental.pallas import tpu_sc as plsc

D_MODEL = 64
NC, NS = 2, 16
NW = NC * NS
CHUNK = 128


@functools.lru_cache(maxsize=None)
def _build(nchunk: int, d: int):
    mesh = plsc.VectorSubcoreMesh(core_axis_name="c", subcore_axis_name="s")

    @functools.partial(
        pl.kernel,
        out_type=jax.ShapeDtypeStruct((NW, nchunk, CHUNK, d), jnp.float32),
        mesh=mesh,
        scratch_types=[
            pltpu.VMEM((nchunk, CHUNK), jnp.int32),
            pltpu.VMEM((CHUNK, d), jnp.float32),
            pltpu.SemaphoreType.DMA,
        ],
        compiler_params=pltpu.CompilerParams(use_tc_tiling_on_sc=False),
    )
    def emb_kernel(x_hbm, table_hbm, out_hbm, idx_v, rows_v, sem):
        wid = lax.axis_index("s") * NC + lax.axis_index("c")
        pltpu.sync_copy(x_hbm.at[wid], idx_v)

        scale = float(d) ** 0.5

        def chunk_body(g, carry):
            pltpu.async_copy(table_hbm.at[idx_v.at[g]], rows_v, sem).wait()

            def row_body(i, c2):
                for j in range(d // 16):
                    sl = pl.ds(j * 16, 16)
                    rows_v[i, sl] = rows_v[i, sl] * scale
                return c2

            lax.fori_loop(0, CHUNK, row_body, 0)
            pltpu.sync_copy(rows_v, out_hbm.at[wid, g])
            return carry

        lax.fori_loop(0, nchunk, chunk_body, 0)

    return emb_kernel


@jax.jit
def kernel(x, table):
    b, t = x.shape
    v, d = table.shape
    total = b * t
    assert total % (NW * CHUNK) == 0 and d % 16 == 0
    nchunk = total // (NW * CHUNK)
    xr = x.reshape(NW, nchunk, CHUNK).astype(jnp.int32)
    out = _build(nchunk, d)(xr, table)
    return out.reshape(b, t, d)
